# baseline (speedup 1.0000x reference)
import jax
import jax.numpy as jnp
from jax import lax
from jax.experimental import pallas as pl
from jax.experimental.pallas import tpu as pltpu
from jax.experimental.pallas import tpu_sc as plsc

D = 64
NC, NS = 2, 16
NW = NC * NS
B = 4096 * 200
C = 128
BPW = B // NW
G = BPW // C


def _emb_body(idx_hbm, tab_hbm, out_hbm, idx_v, buf0, buf1, sem0, sem1):
    w = lax.axis_index("s") * NC + lax.axis_index("c")
    base = w * BPW
    pltpu.sync_copy(idx_hbm.at[pl.ds(w * G, G)], idx_v)

    bufs = (buf0, buf1)
    sems = (sem0, sem1)

    pltpu.async_copy(tab_hbm.at[idx_v.at[0]], buf0, sem0)
    pltpu.async_copy(tab_hbm.at[idx_v.at[1]], buf1, sem1)

    def step(i, carry):
        g = 2 * i
        for b in range(2):
            gc = g + b
            pltpu.make_async_copy(tab_hbm.at[idx_v.at[gc]], bufs[b], sems[b]).wait()
            pltpu.sync_copy(bufs[b], out_hbm.at[pl.ds(base + gc * C, C)])
            pltpu.async_copy(tab_hbm.at[idx_v.at[gc + 2]], bufs[b], sems[b])
        return carry

    lax.fori_loop(0, (G - 2) // 2, step, 0, unroll=False)

    for b in range(2):
        gc = G - 2 + b
        pltpu.make_async_copy(tab_hbm.at[idx_v.at[gc]], bufs[b], sems[b]).wait()
        pltpu.sync_copy(bufs[b], out_hbm.at[pl.ds(base + gc * C, C)])


def kernel(x, embed_weight):
    s0, s1 = x.shape
    xf = x.reshape(-1).astype(jnp.int32).reshape(NW * G, C)
    mesh = plsc.VectorSubcoreMesh(
        core_axis_name="c", subcore_axis_name="s",
        num_cores=NC, num_subcores=NS,
    )
    k = pl.kernel(
        _emb_body,
        out_type=jax.ShapeDtypeStruct((B, D), jnp.float32),
        mesh=mesh,
        scratch_types=[
            pltpu.VMEM((G, C), jnp.int32),
            pltpu.VMEM((C, D), jnp.float32),
            pltpu.VMEM((C, D), jnp.float32),
            pltpu.SemaphoreType.DMA,
            pltpu.SemaphoreType.DMA,
        ],
        compiler_params=pltpu.CompilerParams(use_tc_tiling_on_sc=False),
    )
    out = k(xf, embed_weight)
    return out.reshape(s0, s1, D)


# speedup vs baseline: 1.0246x; 1.0246x over previous
import jax
import jax.numpy as jnp
from jax import lax
from jax.experimental import pallas as pl
from jax.experimental.pallas import tpu as pltpu
from jax.experimental.pallas import tpu_sc as plsc

D = 64
NC, NS = 2, 16
NW = NC * NS
B = 4096 * 200
C = 512
BPW = B // NW
G = BPW // C


def _emb_body(idx_hbm, tab_hbm, out_hbm, idx_v, buf0, buf1, sem0, sem1):
    w = lax.axis_index("s") * NC + lax.axis_index("c")
    base = w * BPW
    pltpu.sync_copy(idx_hbm.at[pl.ds(w * G, G)], idx_v)

    bufs = (buf0, buf1)
    sems = (sem0, sem1)

    pltpu.async_copy(tab_hbm.at[idx_v.at[0]], buf0, sem0)
    pltpu.async_copy(tab_hbm.at[idx_v.at[1]], buf1, sem1)

    def step(i, carry):
        g = 2 * i
        for b in range(2):
            gc = g + b
            pltpu.make_async_copy(tab_hbm.at[idx_v.at[gc]], bufs[b], sems[b]).wait()
            pltpu.sync_copy(bufs[b], out_hbm.at[pl.ds(base + gc * C, C)])
            pltpu.async_copy(tab_hbm.at[idx_v.at[gc + 2]], bufs[b], sems[b])
        return carry

    lax.fori_loop(0, (G - 2) // 2, step, 0, unroll=False)

    for b in range(2):
        gc = G - 2 + b
        pltpu.make_async_copy(tab_hbm.at[idx_v.at[gc]], bufs[b], sems[b]).wait()
        pltpu.sync_copy(bufs[b], out_hbm.at[pl.ds(base + gc * C, C)])


def kernel(x, embed_weight):
    s0, s1 = x.shape
    xf = x.reshape(-1).astype(jnp.int32).reshape(NW * G, C)
    mesh = plsc.VectorSubcoreMesh(
        core_axis_name="c", subcore_axis_name="s",
        num_cores=NC, num_subcores=NS,
    )
    k = pl.kernel(
        _emb_body,
        out_type=jax.ShapeDtypeStruct((B, D), jnp.float32),
        mesh=mesh,
        scratch_types=[
            pltpu.VMEM((G, C), jnp.int32),
            pltpu.VMEM((C, D), jnp.float32),
            pltpu.VMEM((C, D), jnp.float32),
            pltpu.SemaphoreType.DMA,
            pltpu.SemaphoreType.DMA,
        ],
        compiler_params=pltpu.CompilerParams(use_tc_tiling_on_sc=False),
    )
    out = k(xf, embed_weight)
    return out.reshape(s0, s1, D)


# speedup vs baseline: 1.5316x; 1.4948x over previous
import jax
import jax.numpy as jnp
from jax import lax
from jax.experimental import pallas as pl
from jax.experimental.pallas import tpu as pltpu
from jax.experimental.pallas import tpu_sc as plsc

D = 64
DP = 128
NC, NS = 2, 16
NW = NC * NS
B = 4096 * 200
C = 128
BPW = B // NW
G = BPW // C


def _emb_body(idx_hbm, tab_hbm, out_hbm, idx_v, buf0, buf1, sem0, sem1):
    w = lax.axis_index("s") * NC + lax.axis_index("c")
    base = w * BPW
    pltpu.sync_copy(idx_hbm.at[pl.ds(w * G, G)], idx_v)

    bufs = (buf0, buf1)
    sems = (sem0, sem1)

    pltpu.async_copy(tab_hbm.at[idx_v.at[0]], buf0, sem0)
    pltpu.async_copy(tab_hbm.at[idx_v.at[1]], buf1, sem1)

    def step(i, carry):
        g = 2 * i
        for b in range(2):
            gc = g + b
            pltpu.make_async_copy(tab_hbm.at[idx_v.at[gc]], bufs[b], sems[b]).wait()
            pltpu.sync_copy(bufs[b], out_hbm.at[pl.ds(base + gc * C, C)])
            pltpu.async_copy(tab_hbm.at[idx_v.at[gc + 2]], bufs[b], sems[b])
        return carry

    lax.fori_loop(0, (G - 2) // 2, step, 0, unroll=False)

    for b in range(2):
        gc = G - 2 + b
        pltpu.make_async_copy(tab_hbm.at[idx_v.at[gc]], bufs[b], sems[b]).wait()
        pltpu.sync_copy(bufs[b], out_hbm.at[pl.ds(base + gc * C, C)])


def kernel(x, embed_weight):
    s0, s1 = x.shape
    xf = x.reshape(-1).astype(jnp.int32).reshape(NW * G, C)
    tab128 = jnp.pad(embed_weight, ((0, 0), (0, DP - D)))
    mesh = plsc.VectorSubcoreMesh(
        core_axis_name="c", subcore_axis_name="s",
        num_cores=NC, num_subcores=NS,
    )
    k = pl.kernel(
        _emb_body,
        out_type=jax.ShapeDtypeStruct((B, DP), jnp.float32),
        mesh=mesh,
        scratch_types=[
            pltpu.VMEM((G, C), jnp.int32),
            pltpu.VMEM((C, DP), jnp.float32),
            pltpu.VMEM((C, DP), jnp.float32),
            pltpu.SemaphoreType.DMA,
            pltpu.SemaphoreType.DMA,
        ],
    )
    out = k(xf, tab128)
    return out.reshape(s0, s1, DP)[..., :D]
